# Initial kernel scaffold; baseline (speedup 1.0000x reference)
import jax
import jax.numpy as jnp
from jax.experimental import pallas as pl


def kernel(x_variable, x_clause, edge_index_pos, edge_index_neg, Wv, bv, Wc, bc, Wl, bl, Wr, Wf, bf):
    raise NotImplementedError("write your pallas kernel here")


# speedup vs baseline: 3.2343x; 3.2343x over previous
import functools

import jax
import jax.numpy as jnp
from jax import lax
from jax.experimental import pallas as pl
from jax.experimental.pallas import tpu as pltpu
from jax.experimental.pallas import tpu_sc as plsc

H = 128
L = 6
N = 10000
E = 320000
ALPHA = 0.1

NPAD = 10240
NCORES = 2
NSUB = 16
NTILES = NCORES * NSUB
CH = 128
NCHUNK = -(-E // (NTILES * CH))
EPT = NCHUNK * CH
EPAD = EPT * NTILES
RPT = NPAD // NSUB
BROW = 2048
NBLK = NPAD // BROW

_mesh = plsc.VectorSubcoreMesh(core_axis_name="c", subcore_axis_name="s")


def _agg_body(vh, ch, srcs, dsts, zeros, o0, o1, o2, o3, acc, sidx, didx, buf):
    c = lax.axis_index("c")
    s = lax.axis_index("s")
    tile = c * NSUB + s
    outs = [o0, o1, o2, o3]
    for a in range(4):
        x = vh if a < 2 else ch
        pltpu.sync_copy(zeros.at[pl.ds(s * RPT, RPT)], acc.at[pl.ds(s * RPT, RPT)])
        plsc.subcore_barrier()
        pltpu.sync_copy(srcs.at[a, tile], sidx)
        pltpu.sync_copy(dsts.at[a, tile], didx)

        def chunk(j, carry):
            pltpu.sync_copy(x.at[sidx.at[j]], buf)
            pltpu.sync_copy(buf, acc.at[didx.at[j]], add=True)
            return carry

        lax.fori_loop(0, NCHUNK, chunk, 0)
        plsc.subcore_barrier()
        pltpu.sync_copy(acc.at[pl.ds(s * RPT, RPT)],
                        outs[a].at[c, pl.ds(s * RPT, RPT)])
        plsc.subcore_barrier()


_agg_call = pl.kernel(
    _agg_body,
    out_type=[jax.ShapeDtypeStruct((NCORES, NPAD, H), jnp.float32)
              for _ in range(4)],
    mesh=_mesh,
    scratch_types=[
        pltpu.VMEM_SHARED((NPAD, H), jnp.float32),
        pltpu.VMEM((NCHUNK, CH), jnp.int32),
        pltpu.VMEM((NCHUNK, CH), jnp.int32),
        pltpu.VMEM((CH, H), jnp.float32),
    ],
)


def _cnt_body(dsts, zeros, ones, o0, o1, o2, o3, acc, didx, ones_buf):
    c = lax.axis_index("c")
    s = lax.axis_index("s")
    tile = c * NSUB + s
    pltpu.sync_copy(ones, ones_buf)
    outs = [o0, o1, o2, o3]
    for a in range(4):
        pltpu.sync_copy(zeros.at[pl.ds(s * RPT, RPT)],
                        acc.at[pl.ds(s * RPT, RPT)])
        plsc.subcore_barrier()
        pltpu.sync_copy(dsts.at[a, tile], didx)

        def chunk(j, carry):
            pltpu.sync_copy(ones_buf, acc.at[didx.at[j]], add=True)
            return carry

        lax.fori_loop(0, NCHUNK, chunk, 0)
        plsc.subcore_barrier()
        pltpu.sync_copy(acc.at[pl.ds(s * RPT, RPT)],
                        outs[a].at[c, pl.ds(s * RPT, RPT)])
        plsc.subcore_barrier()


_cnt_call = pl.kernel(
    _cnt_body,
    out_type=[jax.ShapeDtypeStruct((NCORES, NPAD, H), jnp.float32)
              for _ in range(4)],
    mesh=_mesh,
    scratch_types=[
        pltpu.VMEM_SHARED((NPAD, H), jnp.float32),
        pltpu.VMEM((NCHUNK, CH), jnp.int32),
        pltpu.VMEM((CH, H), jnp.float32),
    ],
)


def _enc_body(xv, xc, Wv, bv, Wc, bc, cpd, cnd, cps, cns,
              vh0_o, ch0_o, ipd_o, ind_o, ips_o, ins_o):
    vh0_o[...] = jax.nn.relu(
        jnp.dot(xv[...], Wv[...], preferred_element_type=jnp.float32,
                precision=lax.Precision.HIGHEST) + bv[...])
    ch0_o[...] = jax.nn.relu(xc[...] * Wc[...] + bc[...])
    for cref, oref in ((cpd, ipd_o), (cnd, ind_o), (cps, ips_o), (cns, ins_o)):
        cnt = cref[0] + cref[1]
        oref[...] = 1.0 / jnp.maximum(cnt, 1.0)


def _make_enc():
    bspec_row = lambda w: pl.BlockSpec((BROW, w), lambda i: (i, 0))
    bspec_full = lambda a, b: pl.BlockSpec((a, b), lambda i: (0, 0))
    bspec_cnt = pl.BlockSpec((NCORES, BROW, 1), lambda i: (0, i, 0))
    return pl.pallas_call(
        _enc_body,
        grid=(NBLK,),
        in_specs=[bspec_row(4), bspec_row(1), bspec_full(4, H),
                  bspec_full(1, H), bspec_full(1, H), bspec_full(1, H),
                  bspec_cnt, bspec_cnt, bspec_cnt, bspec_cnt],
        out_specs=[bspec_row(H), bspec_row(H), bspec_row(1), bspec_row(1),
                   bspec_row(1), bspec_row(1)],
        out_shape=[jax.ShapeDtypeStruct((NPAD, H), jnp.float32),
                   jax.ShapeDtypeStruct((NPAD, H), jnp.float32)]
        + [jax.ShapeDtypeStruct((NPAD, 1), jnp.float32) for _ in range(4)],
    )


def _layer_math(aCp, aCn, aVp, aVn, ipd, ind, ips, ins, vh, ch, vh0, ch0,
                Wl, bl, Wr):
    dot = functools.partial(jnp.dot, preferred_element_type=jnp.float32,
                            precision=lax.Precision.HIGHEST)
    mcp = (aCp[0] + aCp[1]) * ipd[...]
    mcn = (aCn[0] + aCn[1]) * ind[...]
    mvp = (aVp[0] + aVp[1]) * ips[...]
    mvn = (aVn[0] + aVn[1]) * ins[...]
    out_c = (dot(mcp, Wl[0]) + dot(mcn, Wl[1]) + dot(ch[...], Wr[0] + Wr[1])
             + (bl[0:1] + bl[1:2]))
    out_v = (dot(mvp, Wl[2]) + dot(mvn, Wl[3]) + dot(vh[...], Wr[2] + Wr[3])
             + (bl[2:3] + bl[3:4]))
    c_new = jax.nn.relu((1.0 - ALPHA) * out_c + ALPHA * ch0[...] + ch[...])
    v_new = jax.nn.relu((1.0 - ALPHA) * out_v + ALPHA * vh0[...] + vh[...])
    return v_new, c_new


def _layer_body(aCp, aCn, aVp, aVn, ipd, ind, ips, ins, vh, ch, vh0, ch0,
                Wl, bl, Wr, vo, co):
    v_new, c_new = _layer_math(aCp, aCn, aVp, aVn, ipd, ind, ips, ins,
                               vh, ch, vh0, ch0, Wl, bl, Wr)
    vo[...] = v_new
    co[...] = c_new


def _final_body(aCp, aCn, aVp, aVn, ipd, ind, ips, ins, vh, ch, vh0, ch0,
                Wl, bl, Wr, wfT, bf, yo):
    v_new, _ = _layer_math(aCp, aCn, aVp, aVn, ipd, ind, ips, ins,
                           vh, ch, vh0, ch0, Wl, bl, Wr)
    yo[...] = jnp.sum(v_new * wfT[...], axis=1, keepdims=True) + bf[...]


def _make_layer(final):
    bspec_row = pl.BlockSpec((BROW, H), lambda i: (i, 0))
    bspec_agg = pl.BlockSpec((NCORES, BROW, H), lambda i: (0, i, 0))
    bspec_icnt = pl.BlockSpec((BROW, 1), lambda i: (i, 0))
    in_specs = ([bspec_agg] * 4 + [bspec_icnt] * 4 + [bspec_row] * 4
                + [pl.BlockSpec((4, H, H), lambda i: (0, 0, 0)),
                   pl.BlockSpec((4, H), lambda i: (0, 0)),
                   pl.BlockSpec((4, H, H), lambda i: (0, 0, 0))])
    if final:
        in_specs += [pl.BlockSpec((1, H), lambda i: (0, 0)),
                     pl.BlockSpec((1, 1), lambda i: (0, 0))]
        return pl.pallas_call(
            _final_body, grid=(NBLK,), in_specs=in_specs,
            out_specs=[pl.BlockSpec((BROW, 1), lambda i: (i, 0))],
            out_shape=[jax.ShapeDtypeStruct((NPAD, 1), jnp.float32)],
        )
    return pl.pallas_call(
        _layer_body, grid=(NBLK,), in_specs=in_specs,
        out_specs=[bspec_row, bspec_row],
        out_shape=[jax.ShapeDtypeStruct((NPAD, H), jnp.float32),
                   jax.ShapeDtypeStruct((NPAD, H), jnp.float32)],
    )


def kernel(x_variable, x_clause, edge_index_pos, edge_index_neg,
           Wv, bv, Wc, bc, Wl, bl, Wr, Wf, bf):
    f32 = jnp.float32
    ps, pd = edge_index_pos[0], edge_index_pos[1]
    ns, nd = edge_index_neg[0], edge_index_neg[1]

    npad = EPAD - E

    def padi(x, v):
        return jnp.concatenate([x, jnp.full((npad,), v, jnp.int32)])

    srcs = jnp.stack([padi(ps, 0), padi(ns, 0), padi(pd, 0), padi(nd, 0)])
    dsts = jnp.stack([padi(pd, N), padi(nd, N), padi(ps, N), padi(ns, N)])
    srcs = srcs.reshape(4, NTILES, NCHUNK, CH)
    dsts = dsts.reshape(4, NTILES, NCHUNK, CH)

    xv = jnp.pad(x_variable, ((0, NPAD - N), (0, 0)))
    xc = jnp.pad(x_clause, ((0, NPAD - N), (0, 0)))
    zeros = jnp.zeros((NPAD, H), f32)
    ones = jnp.ones((CH, H), f32)

    cnts = _cnt_call(dsts, zeros, ones)
    cnt_slices = [c[:, :, 0:1] for c in cnts]

    vh0, ch0, ipd, ind, ips, ins = _make_enc()(
        xv, xc, Wv, bv.reshape(1, H), Wc, bc.reshape(1, H), *cnt_slices)

    vh, ch = vh0, ch0
    layer_call = _make_layer(False)
    final_call = _make_layer(True)
    for i in range(L):
        a0, a1, a2, a3 = _agg_call(vh, ch, srcs, dsts, zeros)
        args = (a0, a1, a2, a3, ipd, ind, ips, ins, vh, ch, vh0, ch0,
                Wl[i], bl[i], Wr[i])
        if i < L - 1:
            vh, ch = layer_call(*args)
        else:
            (y,) = final_call(*args, Wf.reshape(1, H), bf.reshape(1, 1))
    return y[:N]
